# baseline (speedup 1.0000x reference)
import functools

import jax
import jax.numpy as jnp
from jax import lax
from jax.experimental import pallas as pl
from jax.experimental.pallas import tpu as pltpu
from jax.experimental.pallas import tpu_sc as plsc

NUM_EMB = 100000
DIM = 128
KD = DIM // 16
B_TOTAL = 4096 * 50
NC, NS = 2, 16
NW = NC * NS
RPC = 128
CPW = B_TOTAL // (NW * RPC)
NBUF = 5
PREF = 3


def _rsqrt_nr(x):
    i = lax.bitcast_convert_type(x, jnp.int32)
    i = jnp.int32(0x5F3759DF) - lax.shift_right_arithmetic(i, 1)
    y = lax.bitcast_convert_type(i, jnp.float32)
    for _ in range(3):
        y = y * (1.5 - 0.5 * x * y * y)
    return y


_mesh = plsc.VectorSubcoreMesh(core_axis_name="c", subcore_axis_name="s")


@functools.partial(
    pl.kernel,
    mesh=_mesh,
    out_type=jax.ShapeDtypeStruct((B_TOTAL, DIM), jnp.float32),
    scratch_types=[
        pltpu.VMEM((1, CPW, RPC), jnp.int32),
        pltpu.VMEM((NBUF, RPC, DIM), jnp.float32),
        pltpu.VMEM((16, 17), jnp.float32),
        pltpu.SemaphoreType.DMA((NBUF,)),
        pltpu.SemaphoreType.DMA((NBUF,)),
    ],
    compiler_params=pltpu.CompilerParams(needs_layout_passes=False),
)
def _embed(x_hbm, tab_hbm, out_hbm, idx_v, rows_v, tmp_v, gsem, osem):
    wid = lax.axis_index("s") * NC + lax.axis_index("c")
    out_base = wid * CPW * RPC
    pltpu.sync_copy(x_hbm.at[pl.ds(wid, 1)], idx_v)

    def start_gather(ci, b):
        pltpu.async_copy(tab_hbm.at[idx_v.at[0, ci]], rows_v.at[b], gsem.at[b])

    def wait_gather(ci, b):
        pltpu.make_async_copy(
            tab_hbm.at[idx_v.at[0, ci]], rows_v.at[b], gsem.at[b]
        ).wait()

    def out_slice(ci):
        return out_hbm.at[pl.ds(out_base + ci * RPC, RPC)]

    lanes = lax.iota(jnp.int32, 16)

    def compute(b):
        rows = rows_v.at[b]

        def group(g, carry):
            r0 = g * 16

            def p1(r4, c):
                for u in range(4):
                    rr = r4 * 4 + u
                    r = r0 + rr
                    vs = [rows[r, pl.ds(k * 16, 16)] for k in range(KD)]
                    p = vs[0] * vs[0]
                    for v in vs[1:]:
                        p = p + v * v
                    plsc.store_scatter(
                        tmp_v, [lanes, jnp.full((16,), rr, jnp.int32)], p
                    )
                return c

            lax.fori_loop(0, 4, p1, 0)
            acc = tmp_v[0, pl.ds(0, 16)]
            for l in range(1, 16):
                acc = acc + tmp_v[l, pl.ds(0, 16)]
            scale_vec = _rsqrt_nr(acc * (1.0 / DIM) + 1e-4)
            for l in range(16):
                plsc.store_scatter(
                    tmp_v, [lanes, jnp.full((16,), l, jnp.int32)], scale_vec
                )

            def p2(r4, c):
                for u in range(4):
                    rr = r4 * 4 + u
                    r = r0 + rr
                    s = tmp_v[rr, pl.ds(0, 16)]
                    for k in range(KD):
                        rows[r, pl.ds(k * 16, 16)] = rows[r, pl.ds(k * 16, 16)] * s
                return c

            lax.fori_loop(0, 4, p2, 0)
            return carry

        lax.fori_loop(0, RPC // 16, group, 0)

    for b in range(PREF):
        start_gather(b, b)

    def outer(o, carry):
        for b in range(NBUF):
            ci = o * NBUF + b
            wait_gather(ci, b)
            compute(b)
            pltpu.async_copy(rows_v.at[b], out_slice(ci), osem.at[b])
            cip = ci + PREF
            bp = (b + PREF) % NBUF

            @pl.when(cip < CPW)
            def _():
                @pl.when(cip >= NBUF)
                def _():
                    pltpu.make_async_copy(
                        rows_v.at[bp], out_slice(cip), osem.at[bp]
                    ).wait()

                start_gather(cip, bp)

        return carry

    lax.fori_loop(0, CPW // NBUF, outer, 0)
    for b in range(NBUF):
        pltpu.make_async_copy(rows_v.at[b], out_slice(b), osem.at[b]).wait()


def kernel(x, weight):
    x2 = x.astype(jnp.int32).reshape(NW, CPW, RPC)
    out = _embed(x2, weight)
    return out.reshape(4096, 50, DIM)


# speedup vs baseline: 2.0604x; 2.0604x over previous
import functools

import jax
import jax.numpy as jnp
from jax import lax
from jax.experimental import pallas as pl
from jax.experimental.pallas import tpu as pltpu
from jax.experimental.pallas import tpu_sc as plsc

NUM_EMB = 100000
DIM = 128
KD = DIM // 16
B_TOTAL = 4096 * 50
NC, NS = 2, 16
NW = NC * NS
RPC = 128
CPW = B_TOTAL // (NW * RPC)
NBUF = 5
PREF = 3


def _rsqrt_nr(x):
    i = lax.bitcast_convert_type(x, jnp.int32)
    i = jnp.int32(0x5F3759DF) - lax.shift_right_arithmetic(i, 1)
    y = lax.bitcast_convert_type(i, jnp.float32)
    for _ in range(3):
        y = y * (1.5 - 0.5 * x * y * y)
    return y


_mesh = plsc.VectorSubcoreMesh(core_axis_name="c", subcore_axis_name="s")


@functools.partial(
    pl.kernel,
    mesh=_mesh,
    out_type=jax.ShapeDtypeStruct((B_TOTAL, DIM), jnp.float32),
    scratch_types=[
        pltpu.VMEM((1, CPW, RPC), jnp.int32),
        pltpu.VMEM((NBUF, RPC, DIM), jnp.float32),
        pltpu.VMEM((16, 17), jnp.float32),
        pltpu.SemaphoreType.DMA((NBUF,)),
        pltpu.SemaphoreType.DMA((NBUF,)),
    ],
    compiler_params=pltpu.CompilerParams(needs_layout_passes=False),
)
def _embed(x_hbm, tab_hbm, out_hbm, idx_v, rows_v, tmp_v, gsem, osem):
    wid = lax.axis_index("s") * NC + lax.axis_index("c")
    out_base = wid * CPW * RPC
    pltpu.sync_copy(x_hbm.at[pl.ds(wid, 1)], idx_v)

    def start_gather(ci, b):
        pltpu.async_copy(tab_hbm.at[idx_v.at[0, ci]], rows_v.at[b], gsem.at[b])

    def wait_gather(ci, b):
        pltpu.make_async_copy(
            tab_hbm.at[idx_v.at[0, ci]], rows_v.at[b], gsem.at[b]
        ).wait()

    def out_slice(ci):
        return out_hbm.at[pl.ds(out_base + ci * RPC, RPC)]

    lanes = lax.iota(jnp.int32, 16)

    def _norm_row(rows, r):
        vs = [rows[r, pl.ds(k * 16, 16)] for k in range(KD)]
        acc = vs[0] * vs[0]
        for v in vs[1:]:
            acc = acc + v * v
        s = jnp.sum(acc)
        scale = _rsqrt_nr(s * (1.0 / DIM) + 1e-4)
        for k, v in enumerate(vs):
            rows[r, pl.ds(k * 16, 16)] = v * scale

    UNROLL = 4

    def compute(b):
        rows = rows_v.at[b]

        def rows_body(i, c):
            for u in range(UNROLL):
                _norm_row(rows, i * UNROLL + u)
            return c

        lax.fori_loop(0, RPC // UNROLL, rows_body, 0)

    for b in range(PREF):
        start_gather(b, b)

    def outer(o, carry):
        for b in range(NBUF):
            ci = o * NBUF + b
            wait_gather(ci, b)
            compute(b)
            pltpu.async_copy(rows_v.at[b], out_slice(ci), osem.at[b])
            cip = ci + PREF
            bp = (b + PREF) % NBUF

            @pl.when(cip < CPW)
            def _():
                @pl.when(cip >= NBUF)
                def _():
                    pltpu.make_async_copy(
                        rows_v.at[bp], out_slice(cip), osem.at[bp]
                    ).wait()

                start_gather(cip, bp)

        return carry

    lax.fori_loop(0, CPW // NBUF, outer, 0)
    for b in range(NBUF):
        pltpu.make_async_copy(rows_v.at[b], out_slice(b), osem.at[b]).wait()


def kernel(x, weight):
    x2 = x.astype(jnp.int32).reshape(NW, CPW, RPC)
    out = _embed(x2, weight)
    return out.reshape(4096, 50, DIM)


# speedup vs baseline: 2.2398x; 1.0871x over previous
import functools

import jax
import jax.numpy as jnp
from jax import lax
from jax.experimental import pallas as pl
from jax.experimental.pallas import tpu as pltpu
from jax.experimental.pallas import tpu_sc as plsc

NUM_EMB = 100000
DIM = 128
KD = DIM // 16
B_TOTAL = 4096 * 50
NC, NS = 2, 16
NW = NC * NS
RPC = 128
CPW = B_TOTAL // (NW * RPC)
NBUF = 5
PREF = 3


def _rsqrt_nr(x):
    i = lax.bitcast_convert_type(x, jnp.int32)
    i = jnp.int32(0x5F3759DF) - lax.shift_right_arithmetic(i, 1)
    y = lax.bitcast_convert_type(i, jnp.float32)
    for _ in range(3):
        y = y * (1.5 - 0.5 * x * y * y)
    return y


_mesh = plsc.VectorSubcoreMesh(core_axis_name="c", subcore_axis_name="s")


@functools.partial(
    pl.kernel,
    mesh=_mesh,
    out_type=jax.ShapeDtypeStruct((B_TOTAL, DIM), jnp.float32),
    scratch_types=[
        pltpu.VMEM((1, CPW, RPC), jnp.int32),
        pltpu.VMEM((NBUF, RPC, DIM), jnp.float32),
        pltpu.VMEM((16, 17), jnp.float32),
        pltpu.SemaphoreType.DMA((NBUF,)),
        pltpu.SemaphoreType.DMA((NBUF,)),
    ],
    compiler_params=pltpu.CompilerParams(needs_layout_passes=False),
)
def _embed(x_hbm, tab_hbm, out_hbm, idx_v, rows_v, tmp_v, gsem, osem):
    wid = lax.axis_index("s") * NC + lax.axis_index("c")
    out_base = wid * CPW * RPC
    pltpu.sync_copy(x_hbm.at[pl.ds(wid, 1)], idx_v)

    def start_gather(ci, b):
        pltpu.async_copy(tab_hbm.at[idx_v.at[0, ci]], rows_v.at[b], gsem.at[b])

    def wait_gather(ci, b):
        pltpu.make_async_copy(
            tab_hbm.at[idx_v.at[0, ci]], rows_v.at[b], gsem.at[b]
        ).wait()

    def out_slice(ci):
        return out_hbm.at[pl.ds(out_base + ci * RPC, RPC)]

    lanes = lax.iota(jnp.int32, 16)

    def _norm_row(rows, r):
        vs = [rows[r, pl.ds(k * 16, 16)] for k in range(KD)]
        acc = vs[0] * vs[0]
        for v in vs[1:]:
            acc = acc + v * v
        s = jnp.sum(acc)
        scale = _rsqrt_nr(jnp.full((16,), s * (1.0 / DIM) + 1e-4, jnp.float32))
        for k, v in enumerate(vs):
            rows[r, pl.ds(k * 16, 16)] = v * scale

    UNROLL = 8

    def compute(b):
        rows = rows_v.at[b]

        def rows_body(i, c):
            for u in range(UNROLL):
                _norm_row(rows, i * UNROLL + u)
            return c

        lax.fori_loop(0, RPC // UNROLL, rows_body, 0)

    for b in range(PREF):
        start_gather(b, b)

    def outer(o, carry):
        for b in range(NBUF):
            ci = o * NBUF + b
            wait_gather(ci, b)
            compute(b)
            pltpu.async_copy(rows_v.at[b], out_slice(ci), osem.at[b])
            cip = ci + PREF
            bp = (b + PREF) % NBUF

            @pl.when(cip < CPW)
            def _():
                @pl.when(cip >= NBUF)
                def _():
                    pltpu.make_async_copy(
                        rows_v.at[bp], out_slice(cip), osem.at[bp]
                    ).wait()

                start_gather(cip, bp)

        return carry

    lax.fori_loop(0, CPW // NBUF, outer, 0)
    for b in range(NBUF):
        pltpu.make_async_copy(rows_v.at[b], out_slice(b), osem.at[b]).wait()


def kernel(x, weight):
    x2 = x.astype(jnp.int32).reshape(NW, CPW, RPC)
    out = _embed(x2, weight)
    return out.reshape(4096, 50, DIM)
